# Initial kernel scaffold; baseline (speedup 1.0000x reference)
import jax
import jax.numpy as jnp
from jax.experimental import pallas as pl


def kernel(x_user, x_item, schema_x, params, edge_index_ui, edge_index_iu, schema_edge_index):
    raise NotImplementedError("write your pallas kernel here")


# speedup vs baseline: 2.2690x; 2.2690x over previous
import jax
import jax.numpy as jnp
from jax import lax
from jax.experimental import pallas as pl
from jax.experimental.pallas import tpu as pltpu
from jax.experimental.pallas import tpu_sc as plsc

N_USER = 10000
N_ITEM = 10000
CH = 128
SH = 64
NB = 8
E = 320000
NUM_LAYERS = 2

NS = 16
CHUNK = 128
K = 160
E_PAD = NS * K * CHUNK
N_PAD = 10112
ROWS_PER_TILE = N_PAD // NS


def _segsum_body(with_counts, *refs):
    if with_counts:
        (xu_t, xi_t, si_ui, di_ui, si_iu, di_iu, zrows_hbm, orows_hbm,
         sui_out, siu_out, cui_out, ciu_out,
         sidx_v, didx_v, rows_v, acc_sh, sem) = refs
    else:
        (xu_t, xi_t, si_ui, di_ui, si_iu, di_iu, zrows_hbm,
         sui_out, siu_out,
         sidx_v, didx_v, rows_v, acc_sh, sem) = refs
        orows_hbm = cui_out = ciu_out = None

    cid = lax.axis_index("c")
    sid = lax.axis_index("s")
    base = sid * ROWS_PER_TILE
    nfull = ROWS_PER_TILE // CHUNK
    rem = ROWS_PER_TILE % CHUNK

    def zero_acc():
        pltpu.sync_copy(zrows_hbm, rows_v)
        for r in range(nfull):
            pltpu.sync_copy(rows_v, acc_sh.at[pl.ds(base + r * CHUNK, CHUNK)])
        if rem:
            off = base + nfull * CHUNK
            pltpu.sync_copy(rows_v.at[pl.ds(0, rem)],
                            acc_sh.at[pl.ds(off, rem)])

    def edge_loop(table_hbm, si_h, di_h, gather):
        def step(j, _):
            pltpu.sync_copy(di_h.at[sid, j], didx_v)
            if gather:
                pltpu.sync_copy(si_h.at[sid, j], sidx_v)
                pltpu.async_copy(table_hbm.at[sidx_v], rows_v, sem).wait()
            pltpu.sync_copy(rows_v, acc_sh.at[didx_v], add=True)
            return 0

        lax.fori_loop(0, K, step, 0)

    def copy_out(out_h):
        for r in range(nfull):
            pltpu.sync_copy(acc_sh.at[pl.ds(base + r * CHUNK, CHUNK)], rows_v)
            pltpu.sync_copy(rows_v, out_h.at[pl.ds(base + r * CHUNK, CHUNK)])
        if rem:
            off = base + nfull * CHUNK
            pltpu.sync_copy(acc_sh.at[pl.ds(off, rem)],
                            rows_v.at[pl.ds(0, rem)])
            pltpu.sync_copy(rows_v.at[pl.ds(0, rem)], out_h.at[pl.ds(off, rem)])

    zero_acc()
    plsc.subcore_barrier()

    @pl.when(cid == 0)
    def _():
        edge_loop(xu_t, si_ui, di_ui, True)

    @pl.when(cid == 1)
    def _():
        edge_loop(xi_t, si_iu, di_iu, True)

    plsc.subcore_barrier()

    @pl.when(cid == 0)
    def _():
        copy_out(sui_out)

    @pl.when(cid == 1)
    def _():
        copy_out(siu_out)

    if with_counts:
        plsc.subcore_barrier()
        zero_acc()
        plsc.subcore_barrier()
        pltpu.sync_copy(orows_hbm, rows_v)

        @pl.when(cid == 0)
        def _():
            edge_loop(None, None, di_ui, False)

        @pl.when(cid == 1)
        def _():
            edge_loop(None, None, di_iu, False)

        plsc.subcore_barrier()

        @pl.when(cid == 0)
        def _():
            copy_out(cui_out)

        @pl.when(cid == 1)
        def _():
            copy_out(ciu_out)


def _make_segsum(with_counts):
    import functools
    mesh = plsc.VectorSubcoreMesh(core_axis_name="c", subcore_axis_name="s")
    n_out = 4 if with_counts else 2
    return pl.kernel(
        functools.partial(_segsum_body, with_counts),
        out_type=tuple(jax.ShapeDtypeStruct((N_PAD, CH), jnp.float32)
                       for _ in range(n_out)),
        mesh=mesh,
        scratch_types=[
            pltpu.VMEM((CHUNK,), jnp.int32),
            pltpu.VMEM((CHUNK,), jnp.int32),
            pltpu.VMEM((CHUNK, CH), jnp.float32),
            pltpu.VMEM_SHARED((N_PAD, CH), jnp.float32),
            pltpu.SemaphoreType.DMA,
        ],
    )


def _schema_body(sx_ref, preW_ref, gcnW_ref, coeffW_ref, aux_ref,
                 b0_ref, b1_ref, ei_ref,
                 sch_ref, ori_ref, w_ui0_ref, w_iu0_ref, w_ui1_ref, w_iu1_ref):
    f32 = jnp.float32
    dn = (((1,), (1,)), ((), ()))
    sx = sx_ref[...]
    h = lax.dot_general(sx, preW_ref[...], dn, preferred_element_type=f32)
    h = h + aux_ref[0:1, :SH]
    ori_ref[...] = h
    x = lax.dot_general(h, gcnW_ref[...], dn, preferred_element_type=f32)

    a = [[1.0, 0.0], [0.0, 1.0]]
    for e in range(4):
        s_e = ei_ref[0, e]
        d_e = ei_ref[1, e]
        for i in range(2):
            for j in range(2):
                hit = jnp.logical_and(d_e == i, s_e == j)
                a[i][j] = a[i][j] + jnp.where(hit, 1.0, 0.0)
    deg = [a[0][0] + a[0][1], a[1][0] + a[1][1]]
    dinv = [lax.rsqrt(deg[0]), lax.rsqrt(deg[1])]
    n = [[dinv[i] * a[i][j] * dinv[j] for j in range(2)] for i in range(2)]

    gb = aux_ref[1:2, :SH]
    row0 = n[0][0] * x[0:1, :] + n[0][1] * x[1:2, :] + gb
    row1 = n[1][0] * x[0:1, :] + n[1][1] * x[1:2, :] + gb
    sch = jnp.maximum(jnp.concatenate([row0, row1, jnp.zeros((6, SH), f32)], 0),
                      0.0)
    sch_ref[...] = sch

    cb = aux_ref[2:3, :NB]
    z_ui = jnp.concatenate([sch[0:1, :], sch[1:2, :]], axis=1)
    z_iu = jnp.concatenate([sch[1:2, :], sch[0:1, :]], axis=1)
    c_ui = lax.dot_general(z_ui, coeffW_ref[...], dn,
                           preferred_element_type=f32) + cb
    c_iu = lax.dot_general(z_iu, coeffW_ref[...], dn,
                           preferred_element_type=f32) + cb

    for b_ref, c, w_ref in ((b0_ref, c_ui, w_ui0_ref),
                            (b0_ref, c_iu, w_iu0_ref),
                            (b1_ref, c_ui, w_ui1_ref),
                            (b1_ref, c_iu, w_iu1_ref)):
        acc = jnp.zeros((CH, CH), f32)
        for i in range(NB):
            acc = acc + c[0:1, i:i + 1] * b_ref[i]
        w_ref[...] = acc


def _schema_call(sx, preW, gcnW, coeffW, aux, b0, b1, ei):
    vmem = pl.BlockSpec(memory_space=pltpu.VMEM)
    return pl.pallas_call(
        _schema_body,
        out_shape=(
            jax.ShapeDtypeStruct((8, SH), jnp.float32),
            jax.ShapeDtypeStruct((2, SH), jnp.float32),
            jax.ShapeDtypeStruct((CH, CH), jnp.float32),
            jax.ShapeDtypeStruct((CH, CH), jnp.float32),
            jax.ShapeDtypeStruct((CH, CH), jnp.float32),
            jax.ShapeDtypeStruct((CH, CH), jnp.float32),
        ),
        in_specs=[vmem] * 7 + [pl.BlockSpec(memory_space=pltpu.SMEM)],
        out_specs=(vmem,) * 6,
    )(sx, preW, gcnW, coeffW, aux, b0, b1, ei)


ROW_BLK = 1264


def _update_body(s_ref, cnt_ref, x_ref, w_ref, pp_ref, o_ref):
    f32 = jnp.float32
    cnt = jnp.maximum(cnt_ref[:, 0:1], 1.0)
    mean = s_ref[...] / cnt
    t = mean + x_ref[...]
    y = lax.dot_general(t, w_ref[...], (((1,), (1,)), ((), ())),
                        preferred_element_type=f32)
    y = y + pp_ref[0:1, :]
    mu = jnp.mean(y, axis=1, keepdims=True)
    d = y - mu
    var = jnp.mean(d * d, axis=1, keepdims=True)
    o = d * lax.rsqrt(var + 1e-5) * pp_ref[1:2, :] + pp_ref[2:3, :]
    o_ref[...] = jnp.maximum(o, 0.0)


def _update_call(s, cnt_t, x_dst, w, pp):
    nblk = N_PAD // ROW_BLK
    return pl.pallas_call(
        _update_body,
        grid=(nblk,),
        in_specs=[
            pl.BlockSpec((ROW_BLK, CH), lambda i: (i, 0)),
            pl.BlockSpec((ROW_BLK, CH), lambda i: (i, 0)),
            pl.BlockSpec((ROW_BLK, CH), lambda i: (i, 0)),
            pl.BlockSpec((CH, CH), lambda i: (0, 0)),
            pl.BlockSpec((8, CH), lambda i: (0, 0)),
        ],
        out_specs=pl.BlockSpec((ROW_BLK, CH), lambda i: (i, 0)),
        out_shape=jax.ShapeDtypeStruct((N_PAD, CH), jnp.float32),
    )(s, cnt_t, x_dst, w, pp)


def _prep_edges(ei):
    src = ei[0].astype(jnp.int32)
    dst = ei[1].astype(jnp.int32)
    pad = E_PAD - E
    src = jnp.concatenate([src, jnp.zeros((pad,), jnp.int32)])
    dst = jnp.concatenate([dst, jnp.full((pad,), N_PAD - 1, jnp.int32)])
    return src.reshape(NS, K, CHUNK), dst.reshape(NS, K, CHUNK)


def _pad_rows(x):
    return jnp.concatenate(
        [x, jnp.zeros((N_PAD - x.shape[0], x.shape[1]), x.dtype)], axis=0)


def _pack_pp(b, ln_w, ln_b):
    z = jnp.zeros_like(b)
    return jnp.stack([b, ln_w, ln_b, z, z, z, z, z], axis=0)


def kernel(x_user, x_item, schema_x, params, edge_index_ui, edge_index_iu,
           schema_edge_index):
    f32 = jnp.float32
    p = params

    aux = jnp.zeros((8, CH), f32)
    aux = aux.at[0, :SH].set(p['pre_b'])
    aux = aux.at[1, :SH].set(p['gcn_b'])
    aux = aux.at[2, :NB].set(p['coeff_b'])
    ei_s = schema_edge_index.astype(jnp.int32)
    sch8, ori, w_ui0, w_iu0, w_ui1, w_iu1 = _schema_call(
        schema_x, p['pre_W'], p['gcn_W'], p['coeff_W'], aux,
        p['bases0'], p['bases1'], ei_s)
    sch = sch8[:2, :]

    si_ui, di_ui = _prep_edges(edge_index_ui)
    si_iu, di_iu = _prep_edges(edge_index_iu)
    seg0 = _make_segsum(True)
    seg1 = _make_segsum(False)
    zrows = jnp.zeros((CHUNK, CH), f32)
    orows = jnp.ones((CHUNK, CH), f32)

    xu = _pad_rows(x_user)
    xi = _pad_rows(x_item)
    ws = ((w_ui0, w_iu0), (w_ui1, w_iu1))
    cnt_ui = cnt_iu = None
    for l in range(NUM_LAYERS):
        w_ui, w_iu = ws[l]
        if l == 0:
            s_ui, s_iu, cnt_ui, cnt_iu = seg0(xu, xi, si_ui, di_ui,
                                              si_iu, di_iu, zrows, orows)
        else:
            s_ui, s_iu = seg1(xu, xi, si_ui, di_ui, si_iu, di_iu, zrows)
        new_i = _update_call(s_ui, cnt_ui, xi, w_ui,
                             _pack_pp(p['b%d_ui' % l], p['ln%d_item_w' % l],
                                      p['ln%d_item_b' % l]))
        new_u = _update_call(s_iu, cnt_iu, xu, w_iu,
                             _pack_pp(p['b%d_iu' % l], p['ln%d_user_w' % l],
                                      p['ln%d_user_b' % l]))
        xu, xi = new_u, new_i

    return (xu[:N_USER], xi[:N_ITEM], sch, ori, p['bases0'], p['bases1'])


# speedup vs baseline: 2.7998x; 1.2340x over previous
import jax
import jax.numpy as jnp
from jax import lax
from jax.experimental import pallas as pl
from jax.experimental.pallas import tpu as pltpu
from jax.experimental.pallas import tpu_sc as plsc

N_USER = 10000
N_ITEM = 10000
CH = 128
SH = 64
NB = 8
E = 320000
NUM_LAYERS = 2

NS = 16
CHUNK = 128
K = 160
E_PAD = NS * K * CHUNK
N_PAD = 10112
ROWS_PER_TILE = N_PAD // NS


def _segsum_body(with_counts, *refs):
    if with_counts:
        (xu_t, xi_t, si_ui, di_ui, si_iu, di_iu, zrows_hbm, orows_hbm,
         sui_out, siu_out, cui_out, ciu_out,
         sidx0, sidx1, didx0, didx1, rows0, rows1, acc_sh,
         sem0, sem1) = refs
    else:
        (xu_t, xi_t, si_ui, di_ui, si_iu, di_iu, zrows_hbm,
         sui_out, siu_out,
         sidx0, sidx1, didx0, didx1, rows0, rows1, acc_sh,
         sem0, sem1) = refs
        orows_hbm = cui_out = ciu_out = None
    sidx = (sidx0, sidx1)
    didx = (didx0, didx1)
    rows = (rows0, rows1)
    gsem = (sem0, sem1)
    rows_v = rows0

    cid = lax.axis_index("c")
    sid = lax.axis_index("s")
    base = sid * ROWS_PER_TILE
    nfull = ROWS_PER_TILE // CHUNK
    rem = ROWS_PER_TILE % CHUNK

    def zero_acc():
        pltpu.sync_copy(zrows_hbm, rows_v)
        for r in range(nfull):
            pltpu.sync_copy(rows_v, acc_sh.at[pl.ds(base + r * CHUNK, CHUNK)])
        if rem:
            off = base + nfull * CHUNK
            pltpu.sync_copy(rows_v.at[pl.ds(0, rem)],
                            acc_sh.at[pl.ds(off, rem)])

    def edge_loop(table_hbm, si_h, di_h, gather):
        if gather:
            pltpu.sync_copy(si_h.at[sid, 0], sidx[0])
            pltpu.sync_copy(di_h.at[sid, 0], didx[0])
            pltpu.async_copy(table_hbm.at[sidx[0]], rows[0], gsem[0]).wait()

            @pl.when(1 < K)
            def _():
                pltpu.sync_copy(si_h.at[sid, 1], sidx[1])
                pltpu.sync_copy(di_h.at[sid, 1], didx[1])

            def pair(g, _):
                for b in range(2):
                    j = 2 * g + b
                    nb = 1 - b

                    @pl.when(j + 1 < K)
                    def _():
                        cp = pltpu.async_copy(table_hbm.at[sidx[nb]],
                                              rows[nb], gsem[nb])
                        pltpu.sync_copy(rows[b], acc_sh.at[didx[b]], add=True)

                        @pl.when(j + 2 < K)
                        def _():
                            pltpu.sync_copy(si_h.at[sid, j + 2], sidx[b])
                            pltpu.sync_copy(di_h.at[sid, j + 2], didx[b])

                        cp.wait()

                    @pl.when(j + 1 >= K)
                    def _():
                        pltpu.sync_copy(rows[b], acc_sh.at[didx[b]], add=True)
                return 0

            lax.fori_loop(0, K // 2, pair, 0)
        else:
            pltpu.sync_copy(di_h.at[sid, 0], didx[0])

            def pair(g, _):
                for b in range(2):
                    j = 2 * g + b
                    nb = 1 - b

                    @pl.when(j + 1 < K)
                    def _():
                        pltpu.sync_copy(di_h.at[sid, j + 1], didx[nb])

                    pltpu.sync_copy(rows[0], acc_sh.at[didx[b]], add=True)
                return 0

            lax.fori_loop(0, K // 2, pair, 0)

    def copy_out(out_h):
        for r in range(nfull):
            pltpu.sync_copy(acc_sh.at[pl.ds(base + r * CHUNK, CHUNK)], rows_v)
            pltpu.sync_copy(rows_v, out_h.at[pl.ds(base + r * CHUNK, CHUNK)])
        if rem:
            off = base + nfull * CHUNK
            pltpu.sync_copy(acc_sh.at[pl.ds(off, rem)],
                            rows_v.at[pl.ds(0, rem)])
            pltpu.sync_copy(rows_v.at[pl.ds(0, rem)], out_h.at[pl.ds(off, rem)])

    zero_acc()
    plsc.subcore_barrier()

    @pl.when(cid == 0)
    def _():
        edge_loop(xu_t, si_ui, di_ui, True)

    @pl.when(cid == 1)
    def _():
        edge_loop(xi_t, si_iu, di_iu, True)

    plsc.subcore_barrier()

    @pl.when(cid == 0)
    def _():
        copy_out(sui_out)

    @pl.when(cid == 1)
    def _():
        copy_out(siu_out)

    if with_counts:
        plsc.subcore_barrier()
        zero_acc()
        plsc.subcore_barrier()
        pltpu.sync_copy(orows_hbm, rows_v)

        @pl.when(cid == 0)
        def _():
            edge_loop(None, None, di_ui, False)

        @pl.when(cid == 1)
        def _():
            edge_loop(None, None, di_iu, False)

        plsc.subcore_barrier()

        @pl.when(cid == 0)
        def _():
            copy_out(cui_out)

        @pl.when(cid == 1)
        def _():
            copy_out(ciu_out)


def _make_segsum(with_counts):
    import functools
    mesh = plsc.VectorSubcoreMesh(core_axis_name="c", subcore_axis_name="s")
    n_out = 4 if with_counts else 2
    return pl.kernel(
        functools.partial(_segsum_body, with_counts),
        out_type=tuple(jax.ShapeDtypeStruct((N_PAD, CH), jnp.float32)
                       for _ in range(n_out)),
        mesh=mesh,
        scratch_types=[
            pltpu.VMEM((CHUNK,), jnp.int32),
            pltpu.VMEM((CHUNK,), jnp.int32),
            pltpu.VMEM((CHUNK,), jnp.int32),
            pltpu.VMEM((CHUNK,), jnp.int32),
            pltpu.VMEM((CHUNK, CH), jnp.float32),
            pltpu.VMEM((CHUNK, CH), jnp.float32),
            pltpu.VMEM_SHARED((N_PAD, CH), jnp.float32),
            pltpu.SemaphoreType.DMA,
            pltpu.SemaphoreType.DMA,
        ],
    )


def _schema_body(sx_ref, preW_ref, gcnW_ref, coeffW_ref, aux_ref,
                 b0_ref, b1_ref, ei_ref,
                 sch_ref, ori_ref, w_ui0_ref, w_iu0_ref, w_ui1_ref, w_iu1_ref):
    f32 = jnp.float32
    dn = (((1,), (1,)), ((), ()))
    sx = sx_ref[...]
    h = lax.dot_general(sx, preW_ref[...], dn, preferred_element_type=f32)
    h = h + aux_ref[0:1, :SH]
    ori_ref[...] = h
    x = lax.dot_general(h, gcnW_ref[...], dn, preferred_element_type=f32)

    a = [[1.0, 0.0], [0.0, 1.0]]
    for e in range(4):
        s_e = ei_ref[0, e]
        d_e = ei_ref[1, e]
        for i in range(2):
            for j in range(2):
                hit = jnp.logical_and(d_e == i, s_e == j)
                a[i][j] = a[i][j] + jnp.where(hit, 1.0, 0.0)
    deg = [a[0][0] + a[0][1], a[1][0] + a[1][1]]
    dinv = [lax.rsqrt(deg[0]), lax.rsqrt(deg[1])]
    n = [[dinv[i] * a[i][j] * dinv[j] for j in range(2)] for i in range(2)]

    gb = aux_ref[1:2, :SH]
    row0 = n[0][0] * x[0:1, :] + n[0][1] * x[1:2, :] + gb
    row1 = n[1][0] * x[0:1, :] + n[1][1] * x[1:2, :] + gb
    sch = jnp.maximum(jnp.concatenate([row0, row1, jnp.zeros((6, SH), f32)], 0),
                      0.0)
    sch_ref[...] = sch

    cb = aux_ref[2:3, :NB]
    z_ui = jnp.concatenate([sch[0:1, :], sch[1:2, :]], axis=1)
    z_iu = jnp.concatenate([sch[1:2, :], sch[0:1, :]], axis=1)
    c_ui = lax.dot_general(z_ui, coeffW_ref[...], dn,
                           preferred_element_type=f32) + cb
    c_iu = lax.dot_general(z_iu, coeffW_ref[...], dn,
                           preferred_element_type=f32) + cb

    for b_ref, c, w_ref in ((b0_ref, c_ui, w_ui0_ref),
                            (b0_ref, c_iu, w_iu0_ref),
                            (b1_ref, c_ui, w_ui1_ref),
                            (b1_ref, c_iu, w_iu1_ref)):
        acc = jnp.zeros((CH, CH), f32)
        for i in range(NB):
            acc = acc + c[0:1, i:i + 1] * b_ref[i]
        w_ref[...] = acc


def _schema_call(sx, preW, gcnW, coeffW, aux, b0, b1, ei):
    vmem = pl.BlockSpec(memory_space=pltpu.VMEM)
    return pl.pallas_call(
        _schema_body,
        out_shape=(
            jax.ShapeDtypeStruct((8, SH), jnp.float32),
            jax.ShapeDtypeStruct((2, SH), jnp.float32),
            jax.ShapeDtypeStruct((CH, CH), jnp.float32),
            jax.ShapeDtypeStruct((CH, CH), jnp.float32),
            jax.ShapeDtypeStruct((CH, CH), jnp.float32),
            jax.ShapeDtypeStruct((CH, CH), jnp.float32),
        ),
        in_specs=[vmem] * 7 + [pl.BlockSpec(memory_space=pltpu.SMEM)],
        out_specs=(vmem,) * 6,
    )(sx, preW, gcnW, coeffW, aux, b0, b1, ei)


ROW_BLK = 1264


def _update_body(s_ref, cnt_ref, x_ref, w_ref, pp_ref, o_ref):
    f32 = jnp.float32
    cnt = jnp.maximum(cnt_ref[:, 0:1], 1.0)
    mean = s_ref[...] / cnt
    t = mean + x_ref[...]
    y = lax.dot_general(t, w_ref[...], (((1,), (1,)), ((), ())),
                        preferred_element_type=f32)
    y = y + pp_ref[0:1, :]
    mu = jnp.mean(y, axis=1, keepdims=True)
    d = y - mu
    var = jnp.mean(d * d, axis=1, keepdims=True)
    o = d * lax.rsqrt(var + 1e-5) * pp_ref[1:2, :] + pp_ref[2:3, :]
    o_ref[...] = jnp.maximum(o, 0.0)


def _update_call(s, cnt_t, x_dst, w, pp):
    nblk = N_PAD // ROW_BLK
    return pl.pallas_call(
        _update_body,
        grid=(nblk,),
        in_specs=[
            pl.BlockSpec((ROW_BLK, CH), lambda i: (i, 0)),
            pl.BlockSpec((ROW_BLK, CH), lambda i: (i, 0)),
            pl.BlockSpec((ROW_BLK, CH), lambda i: (i, 0)),
            pl.BlockSpec((CH, CH), lambda i: (0, 0)),
            pl.BlockSpec((8, CH), lambda i: (0, 0)),
        ],
        out_specs=pl.BlockSpec((ROW_BLK, CH), lambda i: (i, 0)),
        out_shape=jax.ShapeDtypeStruct((N_PAD, CH), jnp.float32),
    )(s, cnt_t, x_dst, w, pp)


def _prep_edges(ei):
    src = ei[0].astype(jnp.int32)
    dst = ei[1].astype(jnp.int32)
    pad = E_PAD - E
    src = jnp.concatenate([src, jnp.zeros((pad,), jnp.int32)])
    dst = jnp.concatenate([dst, jnp.full((pad,), N_PAD - 1, jnp.int32)])
    return src.reshape(NS, K, CHUNK), dst.reshape(NS, K, CHUNK)


def _pad_rows(x):
    return jnp.concatenate(
        [x, jnp.zeros((N_PAD - x.shape[0], x.shape[1]), x.dtype)], axis=0)


def _pack_pp(b, ln_w, ln_b):
    z = jnp.zeros_like(b)
    return jnp.stack([b, ln_w, ln_b, z, z, z, z, z], axis=0)


def kernel(x_user, x_item, schema_x, params, edge_index_ui, edge_index_iu,
           schema_edge_index):
    f32 = jnp.float32
    p = params

    aux = jnp.zeros((8, CH), f32)
    aux = aux.at[0, :SH].set(p['pre_b'])
    aux = aux.at[1, :SH].set(p['gcn_b'])
    aux = aux.at[2, :NB].set(p['coeff_b'])
    ei_s = schema_edge_index.astype(jnp.int32)
    sch8, ori, w_ui0, w_iu0, w_ui1, w_iu1 = _schema_call(
        schema_x, p['pre_W'], p['gcn_W'], p['coeff_W'], aux,
        p['bases0'], p['bases1'], ei_s)
    sch = sch8[:2, :]

    si_ui, di_ui = _prep_edges(edge_index_ui)
    si_iu, di_iu = _prep_edges(edge_index_iu)
    seg0 = _make_segsum(True)
    seg1 = _make_segsum(False)
    zrows = jnp.zeros((CHUNK, CH), f32)
    orows = jnp.ones((CHUNK, CH), f32)

    xu = _pad_rows(x_user)
    xi = _pad_rows(x_item)
    ws = ((w_ui0, w_iu0), (w_ui1, w_iu1))
    cnt_ui = cnt_iu = None
    for l in range(NUM_LAYERS):
        w_ui, w_iu = ws[l]
        if l == 0:
            s_ui, s_iu, cnt_ui, cnt_iu = seg0(xu, xi, si_ui, di_ui,
                                              si_iu, di_iu, zrows, orows)
        else:
            s_ui, s_iu = seg1(xu, xi, si_ui, di_ui, si_iu, di_iu, zrows)
        new_i = _update_call(s_ui, cnt_ui, xi, w_ui,
                             _pack_pp(p['b%d_ui' % l], p['ln%d_item_w' % l],
                                      p['ln%d_item_b' % l]))
        new_u = _update_call(s_iu, cnt_iu, xu, w_iu,
                             _pack_pp(p['b%d_iu' % l], p['ln%d_user_w' % l],
                                      p['ln%d_user_b' % l]))
        xu, xi = new_u, new_i

    return (xu[:N_USER], xi[:N_ITEM], sch, ori, p['bases0'], p['bases1'])


# speedup vs baseline: 2.8945x; 1.0338x over previous
import jax
import jax.numpy as jnp
from jax import lax
from jax.experimental import pallas as pl
from jax.experimental.pallas import tpu as pltpu
from jax.experimental.pallas import tpu_sc as plsc

N_USER = 10000
N_ITEM = 10000
CH = 128
SH = 64
NB = 8
E = 320000
NUM_LAYERS = 2

NS = 16
CHUNK = 128
K = 160
E_PAD = NS * K * CHUNK
N_PAD = 10112
ROWS_PER_TILE = N_PAD // NS


def _segsum_body(with_counts, *refs):
    if with_counts:
        (xu_t, xi_t, si_ui, di_ui, si_iu, di_iu, zrows_hbm, orows_hbm,
         sui_out, siu_out, cui_out, ciu_out,
         sidx0, sidx1, didx0, didx1, didx2, didx3, rows0, rows1, acc_sh,
         sem0, sem1, ssem) = refs
    else:
        (xu_t, xi_t, si_ui, di_ui, si_iu, di_iu, zrows_hbm,
         sui_out, siu_out,
         sidx0, sidx1, didx0, didx1, didx2, didx3, rows0, rows1, acc_sh,
         sem0, sem1, ssem) = refs
        orows_hbm = cui_out = ciu_out = None
    sidx = (sidx0, sidx1)
    didx = (didx0, didx1, didx2, didx3)
    rows = (rows0, rows1)
    gsem = (sem0, sem1)
    rows_v = rows0

    cid = lax.axis_index("c")
    sid = lax.axis_index("s")
    base = sid * ROWS_PER_TILE
    nfull = ROWS_PER_TILE // CHUNK
    rem = ROWS_PER_TILE % CHUNK

    def zero_acc():
        pltpu.sync_copy(zrows_hbm, rows_v)
        for r in range(nfull):
            pltpu.sync_copy(rows_v, acc_sh.at[pl.ds(base + r * CHUNK, CHUNK)])
        if rem:
            off = base + nfull * CHUNK
            pltpu.sync_copy(rows_v.at[pl.ds(0, rem)],
                            acc_sh.at[pl.ds(off, rem)])

    def edge_loop(table_hbm, si_h, di_h, gather):
        if gather:
            pltpu.sync_copy(si_h.at[sid, 0], sidx[0])
            pltpu.sync_copy(di_h.at[sid, 0], didx[0])
            pltpu.async_copy(table_hbm.at[sidx[0]], rows[0], gsem[0]).wait()
            pltpu.sync_copy(si_h.at[sid, 1], sidx[1])
            pltpu.sync_copy(di_h.at[sid, 1], didx[1])

            def quad(t, _):
                for u in range(4):
                    j = 4 * t + u
                    b = u & 1
                    nb = 1 - b
                    d = u
                    nd = (u + 2) % 4

                    @pl.when(j + 1 < K)
                    def _():
                        cp = pltpu.async_copy(table_hbm.at[sidx[nb]],
                                              rows[nb], gsem[nb])
                        sc = pltpu.async_copy(rows[b], acc_sh.at[didx[d]],
                                              ssem, add=True)

                        @pl.when(j + 2 < K)
                        def _():
                            pltpu.sync_copy(si_h.at[sid, j + 2], sidx[b])
                            pltpu.sync_copy(di_h.at[sid, j + 2], didx[nd])

                        sc.wait()
                        cp.wait()

                    @pl.when(j + 1 >= K)
                    def _():
                        pltpu.sync_copy(rows[b], acc_sh.at[didx[d]], add=True)
                return 0

            lax.fori_loop(0, K // 4, quad, 0)
        else:
            pltpu.sync_copy(di_h.at[sid, 0], didx[0])

            def pair(g, _):
                for b in range(2):
                    j = 2 * g + b
                    nb = 1 - b

                    @pl.when(j + 1 < K)
                    def _():
                        pltpu.sync_copy(di_h.at[sid, j + 1], didx[nb])

                    pltpu.sync_copy(rows[0], acc_sh.at[didx[b]], add=True)
                return 0

            lax.fori_loop(0, K // 2, pair, 0)

    def copy_out(out_h):
        for r in range(nfull):
            pltpu.sync_copy(acc_sh.at[pl.ds(base + r * CHUNK, CHUNK)], rows_v)
            pltpu.sync_copy(rows_v, out_h.at[pl.ds(base + r * CHUNK, CHUNK)])
        if rem:
            off = base + nfull * CHUNK
            pltpu.sync_copy(acc_sh.at[pl.ds(off, rem)],
                            rows_v.at[pl.ds(0, rem)])
            pltpu.sync_copy(rows_v.at[pl.ds(0, rem)], out_h.at[pl.ds(off, rem)])

    zero_acc()
    plsc.subcore_barrier()

    @pl.when(cid == 0)
    def _():
        edge_loop(xu_t, si_ui, di_ui, True)

    @pl.when(cid == 1)
    def _():
        edge_loop(xi_t, si_iu, di_iu, True)

    plsc.subcore_barrier()

    @pl.when(cid == 0)
    def _():
        copy_out(sui_out)

    @pl.when(cid == 1)
    def _():
        copy_out(siu_out)

    if with_counts:
        plsc.subcore_barrier()
        zero_acc()
        plsc.subcore_barrier()
        pltpu.sync_copy(orows_hbm, rows_v)

        @pl.when(cid == 0)
        def _():
            edge_loop(None, None, di_ui, False)

        @pl.when(cid == 1)
        def _():
            edge_loop(None, None, di_iu, False)

        plsc.subcore_barrier()

        @pl.when(cid == 0)
        def _():
            copy_out(cui_out)

        @pl.when(cid == 1)
        def _():
            copy_out(ciu_out)


def _make_segsum(with_counts):
    import functools
    mesh = plsc.VectorSubcoreMesh(core_axis_name="c", subcore_axis_name="s")
    n_out = 4 if with_counts else 2
    return pl.kernel(
        functools.partial(_segsum_body, with_counts),
        out_type=tuple(jax.ShapeDtypeStruct((N_PAD, CH), jnp.float32)
                       for _ in range(n_out)),
        mesh=mesh,
        scratch_types=[
            pltpu.VMEM((CHUNK,), jnp.int32),
            pltpu.VMEM((CHUNK,), jnp.int32),
            pltpu.VMEM((CHUNK,), jnp.int32),
            pltpu.VMEM((CHUNK,), jnp.int32),
            pltpu.VMEM((CHUNK,), jnp.int32),
            pltpu.VMEM((CHUNK,), jnp.int32),
            pltpu.VMEM((CHUNK, CH), jnp.float32),
            pltpu.VMEM((CHUNK, CH), jnp.float32),
            pltpu.VMEM_SHARED((N_PAD, CH), jnp.float32),
            pltpu.SemaphoreType.DMA,
            pltpu.SemaphoreType.DMA,
            pltpu.SemaphoreType.DMA,
        ],
    )


def _schema_body(sx_ref, preW_ref, gcnW_ref, coeffW_ref, aux_ref,
                 b0_ref, b1_ref, ei_ref,
                 sch_ref, ori_ref, w_ui0_ref, w_iu0_ref, w_ui1_ref, w_iu1_ref):
    f32 = jnp.float32
    dn = (((1,), (1,)), ((), ()))
    sx = sx_ref[...]
    h = lax.dot_general(sx, preW_ref[...], dn, preferred_element_type=f32)
    h = h + aux_ref[0:1, :SH]
    ori_ref[...] = h
    x = lax.dot_general(h, gcnW_ref[...], dn, preferred_element_type=f32)

    a = [[1.0, 0.0], [0.0, 1.0]]
    for e in range(4):
        s_e = ei_ref[0, e]
        d_e = ei_ref[1, e]
        for i in range(2):
            for j in range(2):
                hit = jnp.logical_and(d_e == i, s_e == j)
                a[i][j] = a[i][j] + jnp.where(hit, 1.0, 0.0)
    deg = [a[0][0] + a[0][1], a[1][0] + a[1][1]]
    dinv = [lax.rsqrt(deg[0]), lax.rsqrt(deg[1])]
    n = [[dinv[i] * a[i][j] * dinv[j] for j in range(2)] for i in range(2)]

    gb = aux_ref[1:2, :SH]
    row0 = n[0][0] * x[0:1, :] + n[0][1] * x[1:2, :] + gb
    row1 = n[1][0] * x[0:1, :] + n[1][1] * x[1:2, :] + gb
    sch = jnp.maximum(jnp.concatenate([row0, row1, jnp.zeros((6, SH), f32)], 0),
                      0.0)
    sch_ref[...] = sch

    cb = aux_ref[2:3, :NB]
    z_ui = jnp.concatenate([sch[0:1, :], sch[1:2, :]], axis=1)
    z_iu = jnp.concatenate([sch[1:2, :], sch[0:1, :]], axis=1)
    c_ui = lax.dot_general(z_ui, coeffW_ref[...], dn,
                           preferred_element_type=f32) + cb
    c_iu = lax.dot_general(z_iu, coeffW_ref[...], dn,
                           preferred_element_type=f32) + cb

    for b_ref, c, w_ref in ((b0_ref, c_ui, w_ui0_ref),
                            (b0_ref, c_iu, w_iu0_ref),
                            (b1_ref, c_ui, w_ui1_ref),
                            (b1_ref, c_iu, w_iu1_ref)):
        acc = jnp.zeros((CH, CH), f32)
        for i in range(NB):
            acc = acc + c[0:1, i:i + 1] * b_ref[i]
        w_ref[...] = acc


def _schema_call(sx, preW, gcnW, coeffW, aux, b0, b1, ei):
    vmem = pl.BlockSpec(memory_space=pltpu.VMEM)
    return pl.pallas_call(
        _schema_body,
        out_shape=(
            jax.ShapeDtypeStruct((8, SH), jnp.float32),
            jax.ShapeDtypeStruct((2, SH), jnp.float32),
            jax.ShapeDtypeStruct((CH, CH), jnp.float32),
            jax.ShapeDtypeStruct((CH, CH), jnp.float32),
            jax.ShapeDtypeStruct((CH, CH), jnp.float32),
            jax.ShapeDtypeStruct((CH, CH), jnp.float32),
        ),
        in_specs=[vmem] * 7 + [pl.BlockSpec(memory_space=pltpu.SMEM)],
        out_specs=(vmem,) * 6,
    )(sx, preW, gcnW, coeffW, aux, b0, b1, ei)


ROW_BLK = 1264


def _update_body(s_ref, cnt_ref, x_ref, w_ref, pp_ref, o_ref):
    f32 = jnp.float32
    cnt = jnp.maximum(cnt_ref[:, 0:1], 1.0)
    mean = s_ref[...] / cnt
    t = mean + x_ref[...]
    y = lax.dot_general(t, w_ref[...], (((1,), (1,)), ((), ())),
                        preferred_element_type=f32)
    y = y + pp_ref[0:1, :]
    mu = jnp.mean(y, axis=1, keepdims=True)
    d = y - mu
    var = jnp.mean(d * d, axis=1, keepdims=True)
    o = d * lax.rsqrt(var + 1e-5) * pp_ref[1:2, :] + pp_ref[2:3, :]
    o_ref[...] = jnp.maximum(o, 0.0)


def _update_call(s, cnt_t, x_dst, w, pp):
    nblk = N_PAD // ROW_BLK
    return pl.pallas_call(
        _update_body,
        grid=(nblk,),
        in_specs=[
            pl.BlockSpec((ROW_BLK, CH), lambda i: (i, 0)),
            pl.BlockSpec((ROW_BLK, CH), lambda i: (i, 0)),
            pl.BlockSpec((ROW_BLK, CH), lambda i: (i, 0)),
            pl.BlockSpec((CH, CH), lambda i: (0, 0)),
            pl.BlockSpec((8, CH), lambda i: (0, 0)),
        ],
        out_specs=pl.BlockSpec((ROW_BLK, CH), lambda i: (i, 0)),
        out_shape=jax.ShapeDtypeStruct((N_PAD, CH), jnp.float32),
    )(s, cnt_t, x_dst, w, pp)


def _prep_edges(ei):
    src = ei[0].astype(jnp.int32)
    dst = ei[1].astype(jnp.int32)
    pad = E_PAD - E
    src = jnp.concatenate([src, jnp.zeros((pad,), jnp.int32)])
    dst = jnp.concatenate([dst, jnp.full((pad,), N_PAD - 1, jnp.int32)])
    return src.reshape(NS, K, CHUNK), dst.reshape(NS, K, CHUNK)


def _pad_rows(x):
    return jnp.concatenate(
        [x, jnp.zeros((N_PAD - x.shape[0], x.shape[1]), x.dtype)], axis=0)


def _pack_pp(b, ln_w, ln_b):
    z = jnp.zeros_like(b)
    return jnp.stack([b, ln_w, ln_b, z, z, z, z, z], axis=0)


def kernel(x_user, x_item, schema_x, params, edge_index_ui, edge_index_iu,
           schema_edge_index):
    f32 = jnp.float32
    p = params

    aux = jnp.zeros((8, CH), f32)
    aux = aux.at[0, :SH].set(p['pre_b'])
    aux = aux.at[1, :SH].set(p['gcn_b'])
    aux = aux.at[2, :NB].set(p['coeff_b'])
    ei_s = schema_edge_index.astype(jnp.int32)
    sch8, ori, w_ui0, w_iu0, w_ui1, w_iu1 = _schema_call(
        schema_x, p['pre_W'], p['gcn_W'], p['coeff_W'], aux,
        p['bases0'], p['bases1'], ei_s)
    sch = sch8[:2, :]

    si_ui, di_ui = _prep_edges(edge_index_ui)
    si_iu, di_iu = _prep_edges(edge_index_iu)
    seg0 = _make_segsum(True)
    seg1 = _make_segsum(False)
    zrows = jnp.zeros((CHUNK, CH), f32)
    orows = jnp.ones((CHUNK, CH), f32)

    xu = _pad_rows(x_user)
    xi = _pad_rows(x_item)
    ws = ((w_ui0, w_iu0), (w_ui1, w_iu1))
    cnt_ui = cnt_iu = None
    for l in range(NUM_LAYERS):
        w_ui, w_iu = ws[l]
        if l == 0:
            s_ui, s_iu, cnt_ui, cnt_iu = seg0(xu, xi, si_ui, di_ui,
                                              si_iu, di_iu, zrows, orows)
        else:
            s_ui, s_iu = seg1(xu, xi, si_ui, di_ui, si_iu, di_iu, zrows)
        new_i = _update_call(s_ui, cnt_ui, xi, w_ui,
                             _pack_pp(p['b%d_ui' % l], p['ln%d_item_w' % l],
                                      p['ln%d_item_b' % l]))
        new_u = _update_call(s_iu, cnt_iu, xu, w_iu,
                             _pack_pp(p['b%d_iu' % l], p['ln%d_user_w' % l],
                                      p['ln%d_user_b' % l]))
        xu, xi = new_u, new_i

    return (xu[:N_USER], xi[:N_ITEM], sch, ori, p['bases0'], p['bases1'])


# speedup vs baseline: 3.0090x; 1.0396x over previous
import jax
import jax.numpy as jnp
from jax import lax
from jax.experimental import pallas as pl
from jax.experimental.pallas import tpu as pltpu
from jax.experimental.pallas import tpu_sc as plsc

N_USER = 10000
N_ITEM = 10000
CH = 128
SH = 64
NB = 8
E = 320000
NUM_LAYERS = 2

NS = 16
CHUNK = 128
K = 160
E_PAD = NS * K * CHUNK
N_PAD = 10112
ROWS_PER_TILE = N_PAD // NS


def _segsum_body(with_counts, *refs):
    if with_counts:
        (xu_t, xi_t, si_ui, di_ui, si_iu, di_iu, zrows_hbm, orows_hbm,
         sui_out, siu_out, cui_out, ciu_out,
         sidx0, sidx1, didx0, didx1, didx2, didx3, rows0, rows1, acc_sh,
         sem0, sem1, ssem) = refs
    else:
        (xu_t, xi_t, si_ui, di_ui, si_iu, di_iu, zrows_hbm,
         sui_out, siu_out,
         sidx0, sidx1, didx0, didx1, didx2, didx3, rows0, rows1, acc_sh,
         sem0, sem1, ssem) = refs
        orows_hbm = cui_out = ciu_out = None
    sidx = (sidx0, sidx1)
    didx = (didx0, didx1, didx2, didx3)
    rows = (rows0, rows1)
    gsem = (sem0, sem1)
    rows_v = rows0

    cid = lax.axis_index("c")
    sid = lax.axis_index("s")
    base = sid * ROWS_PER_TILE
    nfull = ROWS_PER_TILE // CHUNK
    rem = ROWS_PER_TILE % CHUNK

    def zero_acc():
        pltpu.sync_copy(zrows_hbm, rows_v)
        for r in range(nfull):
            pltpu.sync_copy(rows_v, acc_sh.at[pl.ds(base + r * CHUNK, CHUNK)])
        if rem:
            off = base + nfull * CHUNK
            pltpu.sync_copy(rows_v.at[pl.ds(0, rem)],
                            acc_sh.at[pl.ds(off, rem)])

    def edge_loop(table_hbm, si_h, di_h, gather):
        if gather:
            pltpu.sync_copy(si_h.at[sid, 0], sidx[0])
            pltpu.sync_copy(di_h.at[sid, 0], didx[0])
            pltpu.async_copy(table_hbm.at[sidx[0]], rows[0], gsem[0]).wait()
            pltpu.sync_copy(si_h.at[sid, 1], sidx[1])
            pltpu.sync_copy(di_h.at[sid, 1], didx[1])

            def quad(t, _):
                for u in range(4):
                    j = 4 * t + u
                    b = u & 1
                    nb = 1 - b
                    d = u
                    nd = (u + 2) % 4

                    @pl.when(j + 1 < K)
                    def _():
                        cp = pltpu.async_copy(table_hbm.at[sidx[nb]],
                                              rows[nb], gsem[nb])
                        sc = pltpu.async_copy(rows[b], acc_sh.at[didx[d]],
                                              ssem, add=True)

                        @pl.when(j + 2 < K)
                        def _():
                            pltpu.sync_copy(si_h.at[sid, j + 2], sidx[b])
                            pltpu.sync_copy(di_h.at[sid, j + 2], didx[nd])

                        sc.wait()
                        cp.wait()

                    @pl.when(j + 1 >= K)
                    def _():
                        pltpu.sync_copy(rows[b], acc_sh.at[didx[d]], add=True)
                return 0

            lax.fori_loop(0, K // 4, quad, 0)
        else:
            pltpu.sync_copy(di_h.at[sid, 0], didx[0])

            def pair(g, _):
                for b in range(2):
                    j = 2 * g + b
                    nb = 1 - b

                    @pl.when(j + 1 < K)
                    def _():
                        sc = pltpu.async_copy(rows[0], acc_sh.at[didx[b]],
                                              ssem, add=True)
                        pltpu.sync_copy(di_h.at[sid, j + 1], didx[nb])
                        sc.wait()

                    @pl.when(j + 1 >= K)
                    def _():
                        pltpu.sync_copy(rows[0], acc_sh.at[didx[b]], add=True)
                return 0

            lax.fori_loop(0, K // 2, pair, 0)

    def copy_out(out_h):
        for r in range(nfull):
            pltpu.sync_copy(acc_sh.at[pl.ds(base + r * CHUNK, CHUNK)], rows_v)
            pltpu.sync_copy(rows_v, out_h.at[pl.ds(base + r * CHUNK, CHUNK)])
        if rem:
            off = base + nfull * CHUNK
            pltpu.sync_copy(acc_sh.at[pl.ds(off, rem)],
                            rows_v.at[pl.ds(0, rem)])
            pltpu.sync_copy(rows_v.at[pl.ds(0, rem)], out_h.at[pl.ds(off, rem)])

    zero_acc()
    plsc.subcore_barrier()

    @pl.when(cid == 0)
    def _():
        edge_loop(xu_t, si_ui, di_ui, True)

    @pl.when(cid == 1)
    def _():
        edge_loop(xi_t, si_iu, di_iu, True)

    plsc.subcore_barrier()

    @pl.when(cid == 0)
    def _():
        copy_out(sui_out)

    @pl.when(cid == 1)
    def _():
        copy_out(siu_out)

    if with_counts:
        plsc.subcore_barrier()
        zero_acc()
        plsc.subcore_barrier()
        pltpu.sync_copy(orows_hbm, rows_v)

        @pl.when(cid == 0)
        def _():
            edge_loop(None, None, di_ui, False)

        @pl.when(cid == 1)
        def _():
            edge_loop(None, None, di_iu, False)

        plsc.subcore_barrier()

        @pl.when(cid == 0)
        def _():
            copy_out(cui_out)

        @pl.when(cid == 1)
        def _():
            copy_out(ciu_out)


def _make_segsum(with_counts):
    import functools
    mesh = plsc.VectorSubcoreMesh(core_axis_name="c", subcore_axis_name="s")
    n_out = 4 if with_counts else 2
    return pl.kernel(
        functools.partial(_segsum_body, with_counts),
        out_type=tuple(jax.ShapeDtypeStruct((N_PAD, CH), jnp.float32)
                       for _ in range(n_out)),
        mesh=mesh,
        scratch_types=[
            pltpu.VMEM((CHUNK,), jnp.int32),
            pltpu.VMEM((CHUNK,), jnp.int32),
            pltpu.VMEM((CHUNK,), jnp.int32),
            pltpu.VMEM((CHUNK,), jnp.int32),
            pltpu.VMEM((CHUNK,), jnp.int32),
            pltpu.VMEM((CHUNK,), jnp.int32),
            pltpu.VMEM((CHUNK, CH), jnp.float32),
            pltpu.VMEM((CHUNK, CH), jnp.float32),
            pltpu.VMEM_SHARED((N_PAD, CH), jnp.float32),
            pltpu.SemaphoreType.DMA,
            pltpu.SemaphoreType.DMA,
            pltpu.SemaphoreType.DMA,
        ],
    )


def _schema_body(sx_ref, preW_ref, gcnW_ref, coeffW_ref, aux_ref,
                 b0_ref, b1_ref, ei_ref,
                 sch_ref, ori_ref, w_ui0_ref, w_iu0_ref, w_ui1_ref, w_iu1_ref):
    f32 = jnp.float32
    dn = (((1,), (1,)), ((), ()))
    sx = sx_ref[...]
    h = lax.dot_general(sx, preW_ref[...], dn, preferred_element_type=f32)
    h = h + aux_ref[0:1, :SH]
    ori_ref[...] = h
    x = lax.dot_general(h, gcnW_ref[...], dn, preferred_element_type=f32)

    a = [[1.0, 0.0], [0.0, 1.0]]
    for e in range(4):
        s_e = ei_ref[0, e]
        d_e = ei_ref[1, e]
        for i in range(2):
            for j in range(2):
                hit = jnp.logical_and(d_e == i, s_e == j)
                a[i][j] = a[i][j] + jnp.where(hit, 1.0, 0.0)
    deg = [a[0][0] + a[0][1], a[1][0] + a[1][1]]
    dinv = [lax.rsqrt(deg[0]), lax.rsqrt(deg[1])]
    n = [[dinv[i] * a[i][j] * dinv[j] for j in range(2)] for i in range(2)]

    gb = aux_ref[1:2, :SH]
    row0 = n[0][0] * x[0:1, :] + n[0][1] * x[1:2, :] + gb
    row1 = n[1][0] * x[0:1, :] + n[1][1] * x[1:2, :] + gb
    sch = jnp.maximum(jnp.concatenate([row0, row1, jnp.zeros((6, SH), f32)], 0),
                      0.0)
    sch_ref[...] = sch

    cb = aux_ref[2:3, :NB]
    z_ui = jnp.concatenate([sch[0:1, :], sch[1:2, :]], axis=1)
    z_iu = jnp.concatenate([sch[1:2, :], sch[0:1, :]], axis=1)
    c_ui = lax.dot_general(z_ui, coeffW_ref[...], dn,
                           preferred_element_type=f32) + cb
    c_iu = lax.dot_general(z_iu, coeffW_ref[...], dn,
                           preferred_element_type=f32) + cb

    for b_ref, c, w_ref in ((b0_ref, c_ui, w_ui0_ref),
                            (b0_ref, c_iu, w_iu0_ref),
                            (b1_ref, c_ui, w_ui1_ref),
                            (b1_ref, c_iu, w_iu1_ref)):
        acc = jnp.zeros((CH, CH), f32)
        for i in range(NB):
            acc = acc + c[0:1, i:i + 1] * b_ref[i]
        w_ref[...] = acc


def _schema_call(sx, preW, gcnW, coeffW, aux, b0, b1, ei):
    vmem = pl.BlockSpec(memory_space=pltpu.VMEM)
    return pl.pallas_call(
        _schema_body,
        out_shape=(
            jax.ShapeDtypeStruct((8, SH), jnp.float32),
            jax.ShapeDtypeStruct((2, SH), jnp.float32),
            jax.ShapeDtypeStruct((CH, CH), jnp.float32),
            jax.ShapeDtypeStruct((CH, CH), jnp.float32),
            jax.ShapeDtypeStruct((CH, CH), jnp.float32),
            jax.ShapeDtypeStruct((CH, CH), jnp.float32),
        ),
        in_specs=[vmem] * 7 + [pl.BlockSpec(memory_space=pltpu.SMEM)],
        out_specs=(vmem,) * 6,
    )(sx, preW, gcnW, coeffW, aux, b0, b1, ei)


ROW_BLK = 1264


def _update_body(s_ref, cnt_ref, x_ref, w_ref, pp_ref, o_ref):
    f32 = jnp.float32
    cnt = jnp.maximum(cnt_ref[:, 0:1], 1.0)
    mean = s_ref[...] / cnt
    t = mean + x_ref[...]
    y = lax.dot_general(t, w_ref[...], (((1,), (1,)), ((), ())),
                        preferred_element_type=f32)
    y = y + pp_ref[0:1, :]
    mu = jnp.mean(y, axis=1, keepdims=True)
    d = y - mu
    var = jnp.mean(d * d, axis=1, keepdims=True)
    o = d * lax.rsqrt(var + 1e-5) * pp_ref[1:2, :] + pp_ref[2:3, :]
    o_ref[...] = jnp.maximum(o, 0.0)


def _update_call(s, cnt_t, x_dst, w, pp):
    nblk = N_PAD // ROW_BLK
    return pl.pallas_call(
        _update_body,
        grid=(nblk,),
        in_specs=[
            pl.BlockSpec((ROW_BLK, CH), lambda i: (i, 0)),
            pl.BlockSpec((ROW_BLK, CH), lambda i: (i, 0)),
            pl.BlockSpec((ROW_BLK, CH), lambda i: (i, 0)),
            pl.BlockSpec((CH, CH), lambda i: (0, 0)),
            pl.BlockSpec((8, CH), lambda i: (0, 0)),
        ],
        out_specs=pl.BlockSpec((ROW_BLK, CH), lambda i: (i, 0)),
        out_shape=jax.ShapeDtypeStruct((N_PAD, CH), jnp.float32),
    )(s, cnt_t, x_dst, w, pp)


def _prep_edges(ei):
    src = ei[0].astype(jnp.int32)
    dst = ei[1].astype(jnp.int32)
    pad = E_PAD - E
    src = jnp.concatenate([src, jnp.zeros((pad,), jnp.int32)])
    dst = jnp.concatenate([dst, jnp.full((pad,), N_PAD - 1, jnp.int32)])
    return src.reshape(NS, K, CHUNK), dst.reshape(NS, K, CHUNK)


def _pad_rows(x):
    return jnp.concatenate(
        [x, jnp.zeros((N_PAD - x.shape[0], x.shape[1]), x.dtype)], axis=0)


def _pack_pp(b, ln_w, ln_b):
    z = jnp.zeros_like(b)
    return jnp.stack([b, ln_w, ln_b, z, z, z, z, z], axis=0)


def kernel(x_user, x_item, schema_x, params, edge_index_ui, edge_index_iu,
           schema_edge_index):
    f32 = jnp.float32
    p = params

    aux = jnp.zeros((8, CH), f32)
    aux = aux.at[0, :SH].set(p['pre_b'])
    aux = aux.at[1, :SH].set(p['gcn_b'])
    aux = aux.at[2, :NB].set(p['coeff_b'])
    ei_s = schema_edge_index.astype(jnp.int32)
    sch8, ori, w_ui0, w_iu0, w_ui1, w_iu1 = _schema_call(
        schema_x, p['pre_W'], p['gcn_W'], p['coeff_W'], aux,
        p['bases0'], p['bases1'], ei_s)
    sch = sch8[:2, :]

    si_ui, di_ui = _prep_edges(edge_index_ui)
    si_iu, di_iu = _prep_edges(edge_index_iu)
    seg0 = _make_segsum(True)
    seg1 = _make_segsum(False)
    zrows = jnp.zeros((CHUNK, CH), f32)
    orows = jnp.ones((CHUNK, CH), f32)

    xu = _pad_rows(x_user)
    xi = _pad_rows(x_item)
    ws = ((w_ui0, w_iu0), (w_ui1, w_iu1))
    cnt_ui = cnt_iu = None
    for l in range(NUM_LAYERS):
        w_ui, w_iu = ws[l]
        if l == 0:
            s_ui, s_iu, cnt_ui, cnt_iu = seg0(xu, xi, si_ui, di_ui,
                                              si_iu, di_iu, zrows, orows)
        else:
            s_ui, s_iu = seg1(xu, xi, si_ui, di_ui, si_iu, di_iu, zrows)
        new_i = _update_call(s_ui, cnt_ui, xi, w_ui,
                             _pack_pp(p['b%d_ui' % l], p['ln%d_item_w' % l],
                                      p['ln%d_item_b' % l]))
        new_u = _update_call(s_iu, cnt_iu, xu, w_iu,
                             _pack_pp(p['b%d_iu' % l], p['ln%d_user_w' % l],
                                      p['ln%d_user_b' % l]))
        xu, xi = new_u, new_i

    return (xu[:N_USER], xi[:N_ITEM], sch, ori, p['bases0'], p['bases1'])
